# Initial kernel scaffold; baseline (speedup 1.0000x reference)
import jax
import jax.numpy as jnp
from jax.experimental import pallas as pl


def kernel(x, edge_index, W, b):
    raise NotImplementedError("write your pallas kernel here")


# speedup vs baseline: 97.4698x; 97.4698x over previous
import functools

import jax
import jax.numpy as jnp
from jax import lax
from jax.experimental import pallas as pl
from jax.experimental.pallas import tpu as pltpu
from jax.experimental.pallas import tpu_sc as plsc

N = 10000
E = 320000
D = 128
L = 16
NSUB = 16
NPAD = 10240
RNG = NPAD // NSUB
RB = RNG // L
EC = E // NSUB


def _matvec_body(x_ref, w_ref, o_ref):
    o_ref[...] = jnp.dot(x_ref[...], w_ref[...],
                         preferred_element_type=jnp.float32)


def _rsqrt16(x):
    i = plsc.bitcast(x, jnp.int32)
    i = jnp.int32(0x5F3759DF) - lax.shift_right_arithmetic(i, 1)
    r = plsc.bitcast(i, jnp.float32)
    for _ in range(3):
        r = r * (1.5 - 0.5 * x * r * r)
    return r


def _sc_body(src_ref, dst_ref, y_ref, out_ref,
             src_v, dst_v, u_full, acc, red, yr, ur, disr, zr,
             part_sh, stage_sh):
    tid = lax.axis_index("s")
    base = tid * RNG
    ebase = tid * EC

    zeros16 = jnp.zeros((L,), jnp.float32)
    ones16 = jnp.ones((L,), jnp.float32)

    pltpu.sync_copy(src_ref.at[pl.ds(ebase, EC)], src_v)
    pltpu.sync_copy(dst_ref.at[pl.ds(ebase, EC)], dst_v)
    pltpu.sync_copy(y_ref.at[pl.ds(base, RNG)], yr)

    def zero_acc():
        def zb(i, c):
            acc[pl.ds(i * L, L)] = zeros16
            return c
        lax.fori_loop(0, NPAD // L, zb, 0)

    def publish_and_reduce():
        pltpu.sync_copy(acc, part_sh.at[tid])
        plsc.subcore_barrier()
        for k in range(NSUB):
            pltpu.sync_copy(part_sh.at[k, pl.ds(base, RNG)],
                            red.at[pl.ds(k * RNG, RNG)])

    def sum16(j):
        tot = red[pl.ds(j * L, L)]
        for k in range(1, NSUB):
            tot = tot + red[pl.ds(k * RNG + j * L, L)]
        return tot

    zero_acc()

    def deg_body(i, c):
        d16 = dst_v[pl.ds(i * L, L)]
        plsc.addupdate_scatter(acc, [d16], ones16)
        return c
    lax.fori_loop(0, EC // L, deg_body, 0)

    publish_and_reduce()

    def deg_fin(j, c):
        sl = pl.ds(j * L, L)
        deg = sum16(j) + 1.0
        dis = _rsqrt16(deg)
        disr[sl] = dis
        ur[sl] = dis * yr[sl]
        return c
    lax.fori_loop(0, RB, deg_fin, 0)

    def hop_body(i, c):
        sl = pl.ds(i * L, L)
        s16 = src_v[sl]
        d16 = dst_v[sl]
        vals = plsc.load_gather(u_full, [s16])
        plsc.addupdate_scatter(acc, [d16], vals)
        return c

    for _hop in range(2):
        pltpu.sync_copy(ur, stage_sh.at[pl.ds(base, RNG)])
        plsc.subcore_barrier()
        pltpu.sync_copy(stage_sh, u_full)

        zero_acc()
        lax.fori_loop(0, EC // L, hop_body, 0)
        publish_and_reduce()

        def hop_fin(j, c):
            sl = pl.ds(j * L, L)
            z = disr[sl] * (sum16(j) + ur[sl])
            zr[sl] = z
            ur[sl] = disr[sl] * z
            return c
        lax.fori_loop(0, RB, hop_fin, 0)

    pltpu.sync_copy(zr, out_ref.at[pl.ds(base, RNG)])


@functools.partial(
    pl.kernel,
    mesh=plsc.VectorSubcoreMesh(core_axis_name="c", subcore_axis_name="s"),
    out_type=jax.ShapeDtypeStruct((NPAD,), jnp.float32),
    compiler_params=pltpu.CompilerParams(needs_layout_passes=False),
    scratch_types=[
        pltpu.VMEM((EC,), jnp.int32),
        pltpu.VMEM((EC,), jnp.int32),
        pltpu.VMEM((NPAD,), jnp.float32),
        pltpu.VMEM((NPAD,), jnp.float32),
        pltpu.VMEM((NSUB * RNG,), jnp.float32),
        pltpu.VMEM((RNG,), jnp.float32),
        pltpu.VMEM((RNG,), jnp.float32),
        pltpu.VMEM((RNG,), jnp.float32),
        pltpu.VMEM((RNG,), jnp.float32),
        pltpu.VMEM_SHARED((NSUB, NPAD), jnp.float32),
        pltpu.VMEM_SHARED((NPAD,), jnp.float32),
    ],
)
def _sc_propagate(src_ref, dst_ref, y_ref, out_ref, *scratch):
    _sc_body(src_ref, dst_ref, y_ref, out_ref, *scratch)


def kernel(x, edge_index, W, b):
    y = pl.pallas_call(
        _matvec_body,
        out_shape=jax.ShapeDtypeStruct((N, 1), jnp.float32),
    )(x, W).reshape(-1)
    ypad = jnp.pad(y, (0, NPAD - N))
    outpad = _sc_propagate(edge_index[0], edge_index[1], ypad)
    return outpad[:N] + b[0]


# speedup vs baseline: 139.8003x; 1.4343x over previous
import functools

import jax
import jax.numpy as jnp
from jax import lax
from jax.experimental import pallas as pl
from jax.experimental.pallas import tpu as pltpu
from jax.experimental.pallas import tpu_sc as plsc

N = 10000
E = 320000
D = 128
L = 16
NSUB = 16
NPAD = 10240
RNG = NPAD // NSUB
RB = RNG // L
EC = E // NSUB


def _matvec_body(x_ref, w_ref, o_ref):
    o_ref[...] = jnp.dot(x_ref[...], w_ref[...],
                         preferred_element_type=jnp.float32)


def _rsqrt16(x):
    i = plsc.bitcast(x, jnp.int32)
    i = jnp.int32(0x5F3759DF) - lax.shift_right_arithmetic(i, 1)
    r = plsc.bitcast(i, jnp.float32)
    for _ in range(3):
        r = r * (1.5 - 0.5 * x * r * r)
    return r


def _sc_body(src_ref, dst_ref, y_ref, out_ref,
             src_v, dst_v, u_full, acc, red, yr, ur, disr, zr,
             part_sh, stage_sh):
    tid = lax.axis_index("s")
    base = tid * RNG
    ebase = tid * EC

    zeros16 = jnp.zeros((L,), jnp.float32)
    ones16 = jnp.ones((L,), jnp.float32)

    pltpu.sync_copy(src_ref.at[pl.ds(ebase, EC)], src_v)
    pltpu.sync_copy(dst_ref.at[pl.ds(ebase, EC)], dst_v)
    pltpu.sync_copy(y_ref.at[pl.ds(base, RNG)], yr)

    def zero_acc():
        @plsc.parallel_loop(0, NPAD, L, unroll=8)
        def _zb(i):
            acc[pl.ds(i, L)] = zeros16

    def publish_and_reduce():
        pltpu.sync_copy(acc, part_sh.at[tid])
        plsc.subcore_barrier()
        for k in range(NSUB):
            pltpu.sync_copy(part_sh.at[k, pl.ds(base, RNG)],
                            red.at[pl.ds(k * RNG, RNG)])

    def sum16(j):
        tot = red[pl.ds(j * L, L)]
        for k in range(1, NSUB):
            tot = tot + red[pl.ds(k * RNG + j * L, L)]
        return tot

    zero_acc()

    @plsc.parallel_loop(0, EC, L, unroll=8)
    def _deg_body(i):
        d16 = dst_v[pl.ds(i, L)]
        plsc.addupdate_scatter(acc, [d16], ones16)

    publish_and_reduce()

    @plsc.parallel_loop(0, RB, 1, unroll=4)
    def _deg_fin(j):
        sl = pl.ds(j * L, L)
        deg = sum16(j) + 1.0
        dis = _rsqrt16(deg)
        disr[sl] = dis
        ur[sl] = dis * yr[sl]

    def run_hop_loop():
        @plsc.parallel_loop(0, EC, L, unroll=8)
        def _hop_body(i):
            s16 = src_v[pl.ds(i, L)]
            d16 = dst_v[pl.ds(i, L)]
            vals = plsc.load_gather(u_full, [s16])
            plsc.addupdate_scatter(acc, [d16], vals)

    for _hop in range(2):
        pltpu.sync_copy(ur, stage_sh.at[pl.ds(base, RNG)])
        plsc.subcore_barrier()
        pltpu.sync_copy(stage_sh, u_full)

        zero_acc()
        run_hop_loop()
        publish_and_reduce()

        @plsc.parallel_loop(0, RB, 1, unroll=4)
        def _hop_fin(j):
            sl = pl.ds(j * L, L)
            z = disr[sl] * (sum16(j) + ur[sl])
            zr[sl] = z
            ur[sl] = disr[sl] * z

    pltpu.sync_copy(zr, out_ref.at[pl.ds(base, RNG)])


@functools.partial(
    pl.kernel,
    mesh=plsc.VectorSubcoreMesh(core_axis_name="c", subcore_axis_name="s"),
    out_type=jax.ShapeDtypeStruct((NPAD,), jnp.float32),
    compiler_params=pltpu.CompilerParams(needs_layout_passes=False),
    scratch_types=[
        pltpu.VMEM((EC,), jnp.int32),
        pltpu.VMEM((EC,), jnp.int32),
        pltpu.VMEM((NPAD,), jnp.float32),
        pltpu.VMEM((NPAD,), jnp.float32),
        pltpu.VMEM((NSUB * RNG,), jnp.float32),
        pltpu.VMEM((RNG,), jnp.float32),
        pltpu.VMEM((RNG,), jnp.float32),
        pltpu.VMEM((RNG,), jnp.float32),
        pltpu.VMEM((RNG,), jnp.float32),
        pltpu.VMEM_SHARED((NSUB, NPAD), jnp.float32),
        pltpu.VMEM_SHARED((NPAD,), jnp.float32),
    ],
)
def _sc_propagate(src_ref, dst_ref, y_ref, out_ref, *scratch):
    _sc_body(src_ref, dst_ref, y_ref, out_ref, *scratch)


def kernel(x, edge_index, W, b):
    y = pl.pallas_call(
        _matvec_body,
        out_shape=jax.ShapeDtypeStruct((N, 1), jnp.float32),
    )(x, W).reshape(-1)
    ypad = jnp.pad(y, (0, NPAD - N))
    outpad = _sc_propagate(edge_index[0], edge_index[1], ypad)
    return outpad[:N] + b[0]


# speedup vs baseline: 166.4993x; 1.1910x over previous
import functools

import jax
import jax.numpy as jnp
from jax import lax
from jax.experimental import pallas as pl
from jax.experimental.pallas import tpu as pltpu
from jax.experimental.pallas import tpu_sc as plsc

N = 10000
E = 320000
D = 128
L = 16
NSUB = 16
NPAD = 10240
RNG = NPAD // NSUB
RB = RNG // L
EC = E // NSUB


def _matvec_body(x_ref, w_ref, o_ref):
    o_ref[:N] = jnp.dot(x_ref[...], w_ref[...],
                        preferred_element_type=jnp.float32)
    o_ref[N:] = jnp.zeros((NPAD - N, 1), jnp.float32)


def _rsqrt16(x):
    i = plsc.bitcast(x, jnp.int32)
    i = jnp.int32(0x5F3759DF) - lax.shift_right_arithmetic(i, 1)
    r = plsc.bitcast(i, jnp.float32)
    for _ in range(3):
        r = r * (1.5 - 0.5 * x * r * r)
    return r


def _sc_body(edges_ref, y_ref, b_ref, out_ref,
             src_v, dst_v, u_full, acc, red, yr, ur, disr, zr, b_v,
             part_sh, stage_sh):
    tid = lax.axis_index("s")
    base = tid * RNG
    ebase = tid * EC

    zeros16 = jnp.zeros((L,), jnp.float32)
    ones16 = jnp.ones((L,), jnp.float32)

    pltpu.sync_copy(edges_ref.at[pl.ds(ebase, EC)], src_v)
    pltpu.sync_copy(edges_ref.at[pl.ds(E + ebase, EC)], dst_v)
    pltpu.sync_copy(y_ref.at[pl.ds(base, RNG)], yr)
    pltpu.sync_copy(b_ref, b_v)

    def zero_acc():
        @plsc.parallel_loop(0, NPAD, L, unroll=8)
        def _zb(i):
            acc[pl.ds(i, L)] = zeros16

    def publish_and_reduce():
        pltpu.sync_copy(acc, part_sh.at[tid])
        plsc.subcore_barrier()
        for k in range(NSUB):
            pltpu.sync_copy(part_sh.at[k, pl.ds(base, RNG)],
                            red.at[pl.ds(k * RNG, RNG)])

    def sum16(j):
        tot = red[pl.ds(j * L, L)]
        for k in range(1, NSUB):
            tot = tot + red[pl.ds(k * RNG + j * L, L)]
        return tot

    zero_acc()

    @plsc.parallel_loop(0, EC, L, unroll=8)
    def _deg_body(i):
        d16 = dst_v[pl.ds(i, L)]
        plsc.addupdate_scatter(acc, [d16], ones16)

    publish_and_reduce()

    @plsc.parallel_loop(0, RB, 1, unroll=4)
    def _deg_fin(j):
        sl = pl.ds(j * L, L)
        deg = sum16(j) + 1.0
        dis = _rsqrt16(deg)
        disr[sl] = dis
        ur[sl] = dis * yr[sl]

    def run_hop_loop():
        @plsc.parallel_loop(0, EC, L, unroll=8)
        def _hop_body(i):
            s16 = src_v[pl.ds(i, L)]
            d16 = dst_v[pl.ds(i, L)]
            vals = plsc.load_gather(u_full, [s16])
            plsc.addupdate_scatter(acc, [d16], vals)

    for hop in range(2):
        pltpu.sync_copy(ur, stage_sh.at[pl.ds(base, RNG)])
        plsc.subcore_barrier()
        pltpu.sync_copy(stage_sh, u_full)

        zero_acc()
        run_hop_loop()
        publish_and_reduce()

        bias = (jnp.zeros((L,), jnp.float32) if hop == 0
                else plsc.load_gather(b_v, [jnp.zeros((L,), jnp.int32)]))

        @plsc.parallel_loop(0, RB, 1, unroll=4)
        def _hop_fin(j):
            sl = pl.ds(j * L, L)
            z = disr[sl] * (sum16(j) + ur[sl])
            zr[sl] = z + bias
            ur[sl] = disr[sl] * z

    last = NSUB - 1

    @pl.when(tid < last)
    def _():
        pltpu.sync_copy(zr, out_ref.at[pl.ds(base, RNG)])

    @pl.when(tid == last)
    def _():
        pltpu.sync_copy(zr.at[pl.ds(0, N - last * RNG)],
                        out_ref.at[pl.ds(last * RNG, N - last * RNG)])


@functools.partial(
    pl.kernel,
    mesh=plsc.VectorSubcoreMesh(core_axis_name="c", subcore_axis_name="s"),
    out_type=jax.ShapeDtypeStruct((N,), jnp.float32),
    compiler_params=pltpu.CompilerParams(needs_layout_passes=False),
    scratch_types=[
        pltpu.VMEM((EC,), jnp.int32),
        pltpu.VMEM((EC,), jnp.int32),
        pltpu.VMEM((NPAD,), jnp.float32),
        pltpu.VMEM((NPAD,), jnp.float32),
        pltpu.VMEM((NSUB * RNG,), jnp.float32),
        pltpu.VMEM((RNG,), jnp.float32),
        pltpu.VMEM((RNG,), jnp.float32),
        pltpu.VMEM((RNG,), jnp.float32),
        pltpu.VMEM((RNG,), jnp.float32),
        pltpu.VMEM((1,), jnp.float32),
        pltpu.VMEM_SHARED((NSUB, NPAD), jnp.float32),
        pltpu.VMEM_SHARED((NPAD,), jnp.float32),
    ],
)
def _sc_propagate(edges_ref, y_ref, b_ref, out_ref, *scratch):
    _sc_body(edges_ref, y_ref, b_ref, out_ref, *scratch)


def kernel(x, edge_index, W, b):
    ypad = pl.pallas_call(
        _matvec_body,
        out_shape=jax.ShapeDtypeStruct((NPAD, 1), jnp.float32),
    )(x, W).reshape(-1)
    return _sc_propagate(edge_index.reshape(-1), ypad, b)
